# Initial kernel scaffold; baseline (speedup 1.0000x reference)
import jax
import jax.numpy as jnp
from jax.experimental import pallas as pl


def kernel(adapter_feats, v2p_map, batch_idxs, caption_embed, caption_idx, logit_scale):
    raise NotImplementedError("write your pallas kernel here")


# speedup vs baseline: 10.6547x; 10.6547x over previous
import functools

import jax
import jax.numpy as jnp
from jax import lax
from jax.experimental import pallas as pl
from jax.experimental.pallas import tpu as pltpu
from jax.experimental.pallas import tpu_sc as plsc

N_VOXELS = 100000
N_POINTS = 320000
D = 128
B = 16

NC = 2
NS = 16
L = 16
NW = NC * NS

P_TILE = N_POINTS // NW
CH = 128
CHK = 1280
NCHK = -(-P_TILE // CHK)
TAIL = P_TILE - (NCHK - 1) * CHK

C_BINS = B * N_VOXELS
DUMP_V = N_VOXELS
ZCH = 10016
Z_SLICE = 10 * ZCH
C_TOTAL = NS * Z_SLICE
F_SLICE = C_BINS // NS
FCH = 10000


def _hist_body(v2p_hbm, bat_hbm, out_hbm, vb, bb, fidx, ones, zfb, cs):
    c = lax.axis_index("c")
    s = lax.axis_index("s")
    wid = c * NS + s
    base = wid * P_TILE

    def zbody(i, carry):
        zfb[pl.ds(i * L, L)] = jnp.zeros((L,), jnp.float32)
        return carry

    lax.fori_loop(0, ZCH // L, zbody, 0)
    for k in range(Z_SLICE // ZCH):
        pltpu.sync_copy(zfb, cs.at[pl.ds(s * Z_SLICE + k * ZCH, ZCH)])

    for k in range(CH // L):
        ones[pl.ds(k * L, L)] = jnp.ones((L,), jnp.float32)

    plsc.subcore_barrier()

    for t in range(NCHK):
        n_real = CHK if t < NCHK - 1 else TAIL
        off = base + t * CHK
        pltpu.sync_copy(v2p_hbm.at[pl.ds(off, n_real)], vb.at[pl.ds(0, n_real)])
        pltpu.sync_copy(bat_hbm.at[pl.ds(off, n_real)], bb.at[pl.ds(0, n_real)])
        if n_real < CHK:
            for u in range((CHK - n_real) // L):
                vb[pl.ds(n_real + u * L, L)] = jnp.full((L,), DUMP_V, jnp.int32)
                bb[pl.ds(n_real + u * L, L)] = jnp.zeros((L,), jnp.int32)

        def fbody(j, carry):
            for k in range(CH // L):
                o = j * CH + k * L
                v = vb[pl.ds(o, L)]
                b = bb[pl.ds(o, L)]
                fidx[j, pl.ds(k * L, L)] = v * B + b
            return carry

        lax.fori_loop(0, CHK // CH, fbody, 0)

        def scat(j, carry):
            pltpu.sync_copy(ones, cs.at[fidx.at[j]], add=True)
            return carry

        lax.fori_loop(0, CHK // CH, scat, 0)

    plsc.subcore_barrier()
    for k in range(F_SLICE // FCH):
        pltpu.sync_copy(cs.at[pl.ds(s * F_SLICE + k * FCH, FCH)], zfb.at[pl.ds(0, FCH)])
        pltpu.sync_copy(
            zfb.at[pl.ds(0, FCH)],
            out_hbm.at[pl.ds(c * C_BINS + s * F_SLICE + k * FCH, FCH)],
        )


def _histogram(v2p_map, batch_idxs):
    mesh = plsc.VectorSubcoreMesh(
        core_axis_name="c", subcore_axis_name="s", num_cores=NC, num_subcores=NS
    )
    return pl.kernel(
        _hist_body,
        out_type=jax.ShapeDtypeStruct((NC * C_BINS,), jnp.float32),
        mesh=mesh,
        scratch_types=[
            pltpu.VMEM((CHK,), jnp.int32),
            pltpu.VMEM((CHK,), jnp.int32),
            pltpu.VMEM((CHK // CH, CH), jnp.int32),
            pltpu.VMEM((CH,), jnp.float32),
            pltpu.VMEM((ZCH,), jnp.float32),
            pltpu.VMEM_SHARED((C_TOTAL,), jnp.float32),
        ],
    )(v2p_map, batch_idxs)


KC = 2000
NK = N_VOXELS // KC
_DN0 = (((0,), (0,)), ((), ()))


def _mm_body(c_ref, a_ref, cap_ref, ls_ref, out_ref, acc, cnt):
    i = pl.program_id(0)

    @pl.when(i == 0)
    def _init():
        acc[...] = jnp.zeros_like(acc)
        cnt[...] = jnp.zeros_like(cnt)

    c0 = c_ref[0]
    c1 = c_ref[1]
    ablk = a_ref[...]
    acc[...] += lax.dot_general(
        c0, ablk, _DN0, preferred_element_type=jnp.float32,
        precision=lax.Precision.HIGHEST,
    ) + lax.dot_general(
        c1, ablk, _DN0, preferred_element_type=jnp.float32,
        precision=lax.Precision.HIGHEST,
    )
    cnt[...] += jnp.sum(c0 + c1, axis=0, keepdims=True)

    @pl.when(i == NK - 1)
    def _epilogue():
        eye = (
            lax.broadcasted_iota(jnp.int32, (B, B), 0)
            == lax.broadcasted_iota(jnp.int32, (B, B), 1)
        )
        cnt_b = jnp.broadcast_to(cnt[0:1, :], (B, B))
        cnts = jnp.sum(jnp.where(eye, cnt_b, 0.0), axis=1, keepdims=True)
        pooled = acc[...] / jnp.maximum(cnts, 1.0)
        pn = pooled / jnp.maximum(
            jnp.sqrt(jnp.sum(pooled * pooled, axis=1, keepdims=True)), 1e-12
        )
        cap = cap_ref[...]
        cn = cap / jnp.maximum(
            jnp.sqrt(jnp.sum(cap * cap, axis=1, keepdims=True)), 1e-12
        )
        scale = jnp.exp(ls_ref[0, 0])
        out_ref[...] = (
            lax.dot_general(
                pn,
                cn,
                (((1,), (1,)), ((), ())),
                preferred_element_type=jnp.float32,
                precision=lax.Precision.HIGHEST,
            )
            * scale
        )


def _pool_logits(counts, adapter_feats, caption_embed, ls2d):
    return pl.pallas_call(
        _mm_body,
        grid=(NK,),
        in_specs=[
            pl.BlockSpec((NC, KC, B), lambda i: (0, i, 0)),
            pl.BlockSpec((KC, D), lambda i: (i, 0)),
            pl.BlockSpec((B, D), lambda i: (0, 0)),
            pl.BlockSpec(memory_space=pltpu.SMEM),
        ],
        out_specs=pl.BlockSpec((B, B), lambda i: (0, 0)),
        out_shape=jax.ShapeDtypeStruct((B, B), jnp.float32),
        scratch_shapes=[
            pltpu.VMEM((B, D), jnp.float32),
            pltpu.VMEM((8, B), jnp.float32),
        ],
    )(counts, adapter_feats, caption_embed, ls2d)


def kernel(adapter_feats, v2p_map, batch_idxs, caption_embed, caption_idx, logit_scale):
    del caption_idx
    counts2 = _histogram(v2p_map, batch_idxs)
    counts = counts2.reshape(NC, N_VOXELS, B)
    ls2d = jnp.reshape(logit_scale, (1, 1))
    return _pool_logits(counts, adapter_feats, caption_embed, ls2d)


# speedup vs baseline: 10.8819x; 1.0213x over previous
import functools

import jax
import jax.numpy as jnp
from jax import lax
from jax.experimental import pallas as pl
from jax.experimental.pallas import tpu as pltpu
from jax.experimental.pallas import tpu_sc as plsc

N_VOXELS = 100000
N_POINTS = 320000
D = 128
B = 16

NC = 2
NS = 16
L = 16
NW = NC * NS

P_TILE = N_POINTS // NW
CH = 128
CHK = 1280
NCHK = -(-P_TILE // CHK)
TAIL = P_TILE - (NCHK - 1) * CHK

C_BINS = B * N_VOXELS
DUMP_V = N_VOXELS
ZCH = 10016
Z_SLICE = 10 * ZCH
C_TOTAL = NS * Z_SLICE
F_SLICE = C_BINS // NS
FCH = 10000


def _hist_body(v2p_hbm, bat_hbm, out_hbm, vb, bb, fidx, ones, zfb, cs):
    c = lax.axis_index("c")
    s = lax.axis_index("s")
    wid = c * NS + s
    base = wid * P_TILE

    def zbody(i, carry):
        zfb[pl.ds(i * L, L)] = jnp.zeros((L,), jnp.float32)
        return carry

    lax.fori_loop(0, ZCH // L, zbody, 0)
    for k in range(Z_SLICE // ZCH):
        pltpu.sync_copy(zfb, cs.at[pl.ds(s * Z_SLICE + k * ZCH, ZCH)])

    def obody(j, carry):
        ones[pl.ds(j * L, L)] = jnp.ones((L,), jnp.float32)
        return carry

    lax.fori_loop(0, CHK // L, obody, 0)

    plsc.subcore_barrier()

    for t in range(NCHK):
        n_real = CHK if t < NCHK - 1 else TAIL
        off = base + t * CHK
        pltpu.sync_copy(v2p_hbm.at[pl.ds(off, n_real)], vb.at[pl.ds(0, n_real)])
        pltpu.sync_copy(bat_hbm.at[pl.ds(off, n_real)], bb.at[pl.ds(0, n_real)])
        if n_real < CHK:
            for u in range((CHK - n_real) // L):
                vb[pl.ds(n_real + u * L, L)] = jnp.full((L,), DUMP_V, jnp.int32)
                bb[pl.ds(n_real + u * L, L)] = jnp.zeros((L,), jnp.int32)

        def fbody(j, carry):
            o = j * L
            v = vb[pl.ds(o, L)]
            b = bb[pl.ds(o, L)]
            fidx[pl.ds(o, L)] = v * B + b
            return carry

        lax.fori_loop(0, CHK // L, fbody, 0)

        pltpu.sync_copy(ones, cs.at[fidx], add=True)

    plsc.subcore_barrier()
    for k in range(F_SLICE // FCH):
        pltpu.sync_copy(cs.at[pl.ds(s * F_SLICE + k * FCH, FCH)], zfb.at[pl.ds(0, FCH)])
        pltpu.sync_copy(
            zfb.at[pl.ds(0, FCH)],
            out_hbm.at[pl.ds(c * C_BINS + s * F_SLICE + k * FCH, FCH)],
        )


def _histogram(v2p_map, batch_idxs):
    mesh = plsc.VectorSubcoreMesh(
        core_axis_name="c", subcore_axis_name="s", num_cores=NC, num_subcores=NS
    )
    return pl.kernel(
        _hist_body,
        out_type=jax.ShapeDtypeStruct((NC * C_BINS,), jnp.float32),
        mesh=mesh,
        scratch_types=[
            pltpu.VMEM((CHK,), jnp.int32),
            pltpu.VMEM((CHK,), jnp.int32),
            pltpu.VMEM((CHK,), jnp.int32),
            pltpu.VMEM((CHK,), jnp.float32),
            pltpu.VMEM((ZCH,), jnp.float32),
            pltpu.VMEM_SHARED((C_TOTAL,), jnp.float32),
        ],
    )(v2p_map, batch_idxs)


KC = 2000
NK = N_VOXELS // KC
_DN0 = (((0,), (0,)), ((), ()))


def _mm_body(c_ref, a_ref, cap_ref, ls_ref, out_ref, acc, cnt):
    i = pl.program_id(0)

    @pl.when(i == 0)
    def _init():
        acc[...] = jnp.zeros_like(acc)
        cnt[...] = jnp.zeros_like(cnt)

    c0 = c_ref[0]
    c1 = c_ref[1]
    ablk = a_ref[...]
    acc[...] += lax.dot_general(
        c0, ablk, _DN0, preferred_element_type=jnp.float32,
        precision=lax.Precision.HIGHEST,
    ) + lax.dot_general(
        c1, ablk, _DN0, preferred_element_type=jnp.float32,
        precision=lax.Precision.HIGHEST,
    )
    cnt[...] += jnp.sum(c0 + c1, axis=0, keepdims=True)

    @pl.when(i == NK - 1)
    def _epilogue():
        eye = (
            lax.broadcasted_iota(jnp.int32, (B, B), 0)
            == lax.broadcasted_iota(jnp.int32, (B, B), 1)
        )
        cnt_b = jnp.broadcast_to(cnt[0:1, :], (B, B))
        cnts = jnp.sum(jnp.where(eye, cnt_b, 0.0), axis=1, keepdims=True)
        pooled = acc[...] / jnp.maximum(cnts, 1.0)
        pn = pooled / jnp.maximum(
            jnp.sqrt(jnp.sum(pooled * pooled, axis=1, keepdims=True)), 1e-12
        )
        cap = cap_ref[...]
        cn = cap / jnp.maximum(
            jnp.sqrt(jnp.sum(cap * cap, axis=1, keepdims=True)), 1e-12
        )
        scale = jnp.exp(ls_ref[0, 0])
        out_ref[...] = (
            lax.dot_general(
                pn,
                cn,
                (((1,), (1,)), ((), ())),
                preferred_element_type=jnp.float32,
                precision=lax.Precision.HIGHEST,
            )
            * scale
        )


def _pool_logits(counts, adapter_feats, caption_embed, ls2d):
    return pl.pallas_call(
        _mm_body,
        grid=(NK,),
        in_specs=[
            pl.BlockSpec((NC, KC, B), lambda i: (0, i, 0)),
            pl.BlockSpec((KC, D), lambda i: (i, 0)),
            pl.BlockSpec((B, D), lambda i: (0, 0)),
            pl.BlockSpec(memory_space=pltpu.SMEM),
        ],
        out_specs=pl.BlockSpec((B, B), lambda i: (0, 0)),
        out_shape=jax.ShapeDtypeStruct((B, B), jnp.float32),
        scratch_shapes=[
            pltpu.VMEM((B, D), jnp.float32),
            pltpu.VMEM((8, B), jnp.float32),
        ],
    )(counts, adapter_feats, caption_embed, ls2d)


def kernel(adapter_feats, v2p_map, batch_idxs, caption_embed, caption_idx, logit_scale):
    del caption_idx
    counts2 = _histogram(v2p_map, batch_idxs)
    counts = counts2.reshape(NC, N_VOXELS, B)
    ls2d = jnp.reshape(logit_scale, (1, 1))
    return _pool_logits(counts, adapter_feats, caption_embed, ls2d)


# speedup vs baseline: 15.5707x; 1.4309x over previous
import functools

import jax
import jax.numpy as jnp
from jax import lax
from jax.experimental import pallas as pl
from jax.experimental.pallas import tpu as pltpu
from jax.experimental.pallas import tpu_sc as plsc

N_VOXELS = 100000
N_POINTS = 320000
D = 128
B = 16

NC = 2
NS = 16
L = 16
NW = NC * NS

P_TILE = N_POINTS // NW
CH = 128
CHK = 1280
NCHK = -(-P_TILE // CHK)
TAIL = P_TILE - (NCHK - 1) * CHK

C_BINS = B * N_VOXELS
DUMP_B = B
ZCH = 10016
Z_SLICE = 10 * ZCH
C_TOTAL = NS * Z_SLICE
F_SLICE = C_BINS // NS
FCH = 10000


def _hist_body(v2p_hbm, bat_hbm, out_hbm, vb, bb, fidx, ones, zfb, cs):
    c = lax.axis_index("c")
    s = lax.axis_index("s")
    wid = c * NS + s
    base = wid * P_TILE

    def zbody(i, carry):
        zfb[pl.ds(i * L, L)] = jnp.zeros((L,), jnp.float32)
        return carry

    lax.fori_loop(0, ZCH // L, zbody, 0)
    for k in range(Z_SLICE // ZCH):
        pltpu.sync_copy(zfb, cs.at[pl.ds(s * Z_SLICE + k * ZCH, ZCH)])

    def obody(j, carry):
        ones[pl.ds(j * L, L)] = jnp.ones((L,), jnp.float32)
        return carry

    lax.fori_loop(0, CHK // L, obody, 0)

    plsc.subcore_barrier()

    for t in range(NCHK):
        n_real = CHK if t < NCHK - 1 else TAIL
        off = base + t * CHK
        pltpu.sync_copy(v2p_hbm.at[pl.ds(off, n_real)], vb.at[pl.ds(0, n_real)])
        pltpu.sync_copy(bat_hbm.at[pl.ds(off, n_real)], bb.at[pl.ds(0, n_real)])
        if n_real < CHK:
            for u in range((CHK - n_real) // L):
                vb[pl.ds(n_real + u * L, L)] = jnp.zeros((L,), jnp.int32)
                bb[pl.ds(n_real + u * L, L)] = jnp.full((L,), DUMP_B, jnp.int32)

        def fbody(j, carry):
            o = j * L
            v = vb[pl.ds(o, L)]
            b = bb[pl.ds(o, L)]
            fidx[pl.ds(o, L)] = b * N_VOXELS + v
            return carry

        lax.fori_loop(0, CHK // L, fbody, 0)

        pltpu.sync_copy(ones, cs.at[fidx], add=True)

    plsc.subcore_barrier()
    for k in range(F_SLICE // FCH):
        pltpu.sync_copy(cs.at[pl.ds(s * F_SLICE + k * FCH, FCH)], zfb.at[pl.ds(0, FCH)])
        pltpu.sync_copy(
            zfb.at[pl.ds(0, FCH)],
            out_hbm.at[pl.ds(c * C_BINS + s * F_SLICE + k * FCH, FCH)],
        )


def _histogram(v2p_map, batch_idxs):
    mesh = plsc.VectorSubcoreMesh(
        core_axis_name="c", subcore_axis_name="s", num_cores=NC, num_subcores=NS
    )
    return pl.kernel(
        _hist_body,
        out_type=jax.ShapeDtypeStruct((NC * C_BINS,), jnp.float32),
        mesh=mesh,
        scratch_types=[
            pltpu.VMEM((CHK,), jnp.int32),
            pltpu.VMEM((CHK,), jnp.int32),
            pltpu.VMEM((CHK,), jnp.int32),
            pltpu.VMEM((CHK,), jnp.float32),
            pltpu.VMEM((ZCH,), jnp.float32),
            pltpu.VMEM_SHARED((C_TOTAL,), jnp.float32),
        ],
    )(v2p_map, batch_idxs)


VSTEP = 2048
NKM = 98304 // VSTEP
V_TAIL = N_VOXELS - NKM * VSTEP
_DN_MK = (((1,), (0,)), ((), ()))
_PREC = lax.Precision.HIGHEST


def _mm_main_body(c_ref, a_ref, acc_ref, cnt_ref):
    i = pl.program_id(0)

    @pl.when(i == 0)
    def _init():
        acc_ref[...] = jnp.zeros_like(acc_ref)
        cnt_ref[...] = jnp.zeros_like(cnt_ref)

    c0 = c_ref[0]
    c1 = c_ref[1]
    ablk = a_ref[...]
    acc_ref[...] += lax.dot_general(
        c0, ablk, _DN_MK, preferred_element_type=jnp.float32, precision=_PREC
    ) + lax.dot_general(
        c1, ablk, _DN_MK, preferred_element_type=jnp.float32, precision=_PREC
    )
    cnt_ref[...] += jnp.sum(c0 + c1, axis=1, keepdims=True)


def _mm_tail_body(ct_ref, at_ref, acc_ref, cnt_ref, cap_ref, ls_ref, out_ref):
    at = at_ref[...]
    sums = acc_ref[...] + lax.dot_general(
        ct_ref[0], at, _DN_MK, preferred_element_type=jnp.float32, precision=_PREC
    ) + lax.dot_general(
        ct_ref[1], at, _DN_MK, preferred_element_type=jnp.float32, precision=_PREC
    )
    cnts = cnt_ref[:, 0:1] + jnp.sum(ct_ref[0] + ct_ref[1], axis=1, keepdims=True)
    pooled = sums / jnp.maximum(cnts, 1.0)
    pn = pooled / jnp.maximum(
        jnp.sqrt(jnp.sum(pooled * pooled, axis=1, keepdims=True)), 1e-12
    )
    cap = cap_ref[...]
    cn = cap / jnp.maximum(
        jnp.sqrt(jnp.sum(cap * cap, axis=1, keepdims=True)), 1e-12
    )
    scale = jnp.exp(ls_ref[0, 0])
    out_ref[...] = (
        lax.dot_general(
            pn, cn, (((1,), (1,)), ((), ())),
            preferred_element_type=jnp.float32, precision=_PREC,
        )
        * scale
    )


def _pool_main(counts_bm, adapter_feats):
    return pl.pallas_call(
        _mm_main_body,
        grid=(NKM,),
        in_specs=[
            pl.BlockSpec((NC, B, VSTEP), lambda i: (0, 0, i)),
            pl.BlockSpec((VSTEP, D), lambda i: (i, 0)),
        ],
        out_specs=[
            pl.BlockSpec((B, D), lambda i: (0, 0)),
            pl.BlockSpec((B, D), lambda i: (0, 0)),
        ],
        out_shape=[
            jax.ShapeDtypeStruct((B, D), jnp.float32),
            jax.ShapeDtypeStruct((B, D), jnp.float32),
        ],
    )(counts_bm, adapter_feats)


def _pool_tail(ct, at, acc, cnt, caption_embed, ls2d):
    return pl.pallas_call(
        _mm_tail_body,
        in_specs=[
            pl.BlockSpec((NC, B, V_TAIL), lambda: (0, 0, 0)),
            pl.BlockSpec((V_TAIL, D), lambda: (0, 0)),
            pl.BlockSpec((B, D), lambda: (0, 0)),
            pl.BlockSpec((B, D), lambda: (0, 0)),
            pl.BlockSpec((B, D), lambda: (0, 0)),
            pl.BlockSpec(memory_space=pltpu.SMEM),
        ],
        out_specs=pl.BlockSpec((B, B), lambda: (0, 0)),
        out_shape=jax.ShapeDtypeStruct((B, B), jnp.float32),
    )(ct, at, acc, cnt, caption_embed, ls2d)


def kernel(adapter_feats, v2p_map, batch_idxs, caption_embed, caption_idx, logit_scale):
    del caption_idx
    counts_bm = _histogram(v2p_map, batch_idxs).reshape(NC, B, N_VOXELS)
    acc, cnt = _pool_main(counts_bm, adapter_feats)
    ct = lax.slice(counts_bm, (0, 0, NKM * VSTEP), (NC, B, N_VOXELS))
    at = lax.slice(adapter_feats, (NKM * VSTEP, 0), (N_VOXELS, D))
    ls2d = jnp.reshape(logit_scale, (1, 1))
    return _pool_tail(ct, at, acc, cnt, caption_embed, ls2d)


# speedup vs baseline: 17.7039x; 1.1370x over previous
import functools

import jax
import jax.numpy as jnp
from jax import lax
from jax.experimental import pallas as pl
from jax.experimental.pallas import tpu as pltpu
from jax.experimental.pallas import tpu_sc as plsc

N_VOXELS = 100000
N_POINTS = 320000
D = 128
B = 16

NC = 2
NS = 16
L = 16
NW = NC * NS

P_TILE = N_POINTS // NW
CH = 128
CHK = 1280
NCHK = -(-P_TILE // CHK)
TAIL = P_TILE - (NCHK - 1) * CHK

C_BINS = B * N_VOXELS
DUMP_B = B
ZCH = 10016
Z_SLICE = 10 * ZCH
C_TOTAL = NS * Z_SLICE
F_SLICE = C_BINS // NS
FCH = 10000


def _hist_body(v2p_hbm, bat_hbm, out_hbm, vb, bb, fidx, ones, zfb, cs):
    c = lax.axis_index("c")
    s = lax.axis_index("s")
    wid = c * NS + s
    base = wid * P_TILE

    def zbody(i, carry):
        zfb[pl.ds(i * L, L)] = jnp.zeros((L,), jnp.float32)
        return carry

    lax.fori_loop(0, ZCH // L, zbody, 0)
    for k in range(Z_SLICE // ZCH):
        pltpu.sync_copy(zfb, cs.at[pl.ds(s * Z_SLICE + k * ZCH, ZCH)])

    def obody(j, carry):
        ones[pl.ds(j * L, L)] = jnp.ones((L,), jnp.float32)
        return carry

    lax.fori_loop(0, CHK // L, obody, 0)

    plsc.subcore_barrier()

    for t in range(NCHK):
        n_real = CHK if t < NCHK - 1 else TAIL
        off = base + t * CHK
        pltpu.sync_copy(v2p_hbm.at[pl.ds(off, n_real)], vb.at[pl.ds(0, n_real)])
        pltpu.sync_copy(bat_hbm.at[pl.ds(off, n_real)], bb.at[pl.ds(0, n_real)])
        if n_real < CHK:
            for u in range((CHK - n_real) // L):
                vb[pl.ds(n_real + u * L, L)] = jnp.zeros((L,), jnp.int32)
                bb[pl.ds(n_real + u * L, L)] = jnp.full((L,), DUMP_B, jnp.int32)

        def fbody(j, carry):
            o = j * L
            v = vb[pl.ds(o, L)]
            b = bb[pl.ds(o, L)]
            fidx[pl.ds(o, L)] = b * N_VOXELS + v
            return carry

        lax.fori_loop(0, CHK // L, fbody, 0)

        pltpu.sync_copy(ones, cs.at[fidx], add=True)

    plsc.subcore_barrier()
    for k in range(F_SLICE // FCH):
        pltpu.sync_copy(cs.at[pl.ds(s * F_SLICE + k * FCH, FCH)], zfb.at[pl.ds(0, FCH)])
        pltpu.sync_copy(
            zfb.at[pl.ds(0, FCH)],
            out_hbm.at[pl.ds(c * C_BINS + s * F_SLICE + k * FCH, FCH)],
        )


def _histogram(v2p_map, batch_idxs):
    mesh = plsc.VectorSubcoreMesh(
        core_axis_name="c", subcore_axis_name="s", num_cores=NC, num_subcores=NS
    )
    return pl.kernel(
        _hist_body,
        out_type=jax.ShapeDtypeStruct((NC * C_BINS,), jnp.float32),
        mesh=mesh,
        scratch_types=[
            pltpu.VMEM((CHK,), jnp.int32),
            pltpu.VMEM((CHK,), jnp.int32),
            pltpu.VMEM((CHK,), jnp.int32),
            pltpu.VMEM((CHK,), jnp.float32),
            pltpu.VMEM((ZCH,), jnp.float32),
            pltpu.VMEM_SHARED((C_TOTAL,), jnp.float32),
        ],
    )(v2p_map, batch_idxs)


VSTEP = 2048
NKM = 98304 // VSTEP
V_TAIL = N_VOXELS - NKM * VSTEP
_DN_MK = (((1,), (0,)), ((), ()))
_PREC = lax.Precision.DEFAULT


def _mm_main_body(c_ref, a_ref, acc_ref, cnt_ref):
    i = pl.program_id(0)

    @pl.when(i == 0)
    def _init():
        acc_ref[...] = jnp.zeros_like(acc_ref)
        cnt_ref[...] = jnp.zeros_like(cnt_ref)

    c0 = c_ref[0]
    c1 = c_ref[1]
    ablk = a_ref[...]
    acc_ref[...] += lax.dot_general(
        c0, ablk, _DN_MK, preferred_element_type=jnp.float32, precision=_PREC
    ) + lax.dot_general(
        c1, ablk, _DN_MK, preferred_element_type=jnp.float32, precision=_PREC
    )
    cnt_ref[...] += jnp.sum(c0 + c1, axis=1, keepdims=True)


def _mm_tail_body(ct_ref, at_ref, acc_ref, cnt_ref, cap_ref, ls_ref, out_ref):
    at = at_ref[...]
    sums = acc_ref[...] + lax.dot_general(
        ct_ref[0], at, _DN_MK, preferred_element_type=jnp.float32, precision=_PREC
    ) + lax.dot_general(
        ct_ref[1], at, _DN_MK, preferred_element_type=jnp.float32, precision=_PREC
    )
    cnts = cnt_ref[:, 0:1] + jnp.sum(ct_ref[0] + ct_ref[1], axis=1, keepdims=True)
    pooled = sums / jnp.maximum(cnts, 1.0)
    pn = pooled / jnp.maximum(
        jnp.sqrt(jnp.sum(pooled * pooled, axis=1, keepdims=True)), 1e-12
    )
    cap = cap_ref[...]
    cn = cap / jnp.maximum(
        jnp.sqrt(jnp.sum(cap * cap, axis=1, keepdims=True)), 1e-12
    )
    scale = jnp.exp(ls_ref[0, 0])
    out_ref[...] = (
        lax.dot_general(
            pn, cn, (((1,), (1,)), ((), ())),
            preferred_element_type=jnp.float32, precision=_PREC,
        )
        * scale
    )


def _pool_main(counts_bm, adapter_feats):
    return pl.pallas_call(
        _mm_main_body,
        grid=(NKM,),
        in_specs=[
            pl.BlockSpec((NC, B, VSTEP), lambda i: (0, 0, i)),
            pl.BlockSpec((VSTEP, D), lambda i: (i, 0)),
        ],
        out_specs=[
            pl.BlockSpec((B, D), lambda i: (0, 0)),
            pl.BlockSpec((B, D), lambda i: (0, 0)),
        ],
        out_shape=[
            jax.ShapeDtypeStruct((B, D), jnp.float32),
            jax.ShapeDtypeStruct((B, D), jnp.float32),
        ],
    )(counts_bm, adapter_feats)


def _pool_tail(ct, at, acc, cnt, caption_embed, ls2d):
    return pl.pallas_call(
        _mm_tail_body,
        in_specs=[
            pl.BlockSpec((NC, B, V_TAIL), lambda: (0, 0, 0)),
            pl.BlockSpec((V_TAIL, D), lambda: (0, 0)),
            pl.BlockSpec((B, D), lambda: (0, 0)),
            pl.BlockSpec((B, D), lambda: (0, 0)),
            pl.BlockSpec((B, D), lambda: (0, 0)),
            pl.BlockSpec(memory_space=pltpu.SMEM),
        ],
        out_specs=pl.BlockSpec((B, B), lambda: (0, 0)),
        out_shape=jax.ShapeDtypeStruct((B, B), jnp.float32),
    )(ct, at, acc, cnt, caption_embed, ls2d)


def kernel(adapter_feats, v2p_map, batch_idxs, caption_embed, caption_idx, logit_scale):
    del caption_idx
    counts_bm = _histogram(v2p_map, batch_idxs).reshape(NC, B, N_VOXELS)
    acc, cnt = _pool_main(counts_bm, adapter_feats)
    ct = lax.slice(counts_bm, (0, 0, NKM * VSTEP), (NC, B, N_VOXELS))
    at = lax.slice(adapter_feats, (NKM * VSTEP, 0), (N_VOXELS, D))
    ls2d = jnp.reshape(logit_scale, (1, 1))
    return _pool_tail(ct, at, acc, cnt, caption_embed, ls2d)


# speedup vs baseline: 18.4738x; 1.0435x over previous
import functools

import jax
import jax.numpy as jnp
from jax import lax
from jax.experimental import pallas as pl
from jax.experimental.pallas import tpu as pltpu
from jax.experimental.pallas import tpu_sc as plsc

N_VOXELS = 100000
N_POINTS = 320000
D = 128
B = 16

NC = 2
NS = 16
L = 16
NW = NC * NS

P_TILE = N_POINTS // NW
CH = 128
CHK = 1280
NCHK = -(-P_TILE // CHK)
TAIL = P_TILE - (NCHK - 1) * CHK

C_BINS = B * N_VOXELS
DUMP_B = B
ZCH = 10240
Z_SLICE = 10 * ZCH
C_TOTAL = NS * Z_SLICE
F_SLICE = C_BINS // NS
FCH = 5120
NFCH = -(-F_SLICE // FCH)
F_TAIL = F_SLICE - (NFCH - 1) * FCH


def _hist_body(v2p_hbm, bat_hbm, out_hbm, vb, bb, fidx, ones, zfb, cs,
               sem_a, sem_f0, sem_f1):
    c = lax.axis_index("c")
    s = lax.axis_index("s")
    wid = c * NS + s
    base = wid * P_TILE

    def zbody(i, carry):
        zfb[pl.ds(i * L, L)] = jnp.zeros((L,), jnp.float32)
        return carry

    lax.fori_loop(0, ZCH // L, zbody, 0)
    zero_descs = [
        pltpu.async_copy(zfb, cs.at[pl.ds(s * Z_SLICE + k * ZCH, ZCH)], sem_a)
        for k in range(Z_SLICE // ZCH)
    ]

    def obody(j, carry):
        ones[pl.ds(j * L, L)] = jnp.ones((L,), jnp.float32)
        return carry

    lax.fori_loop(0, CHK // L, obody, 0)

    for t in range(NCHK):
        n_real = CHK if t < NCHK - 1 else TAIL
        off = base + t * CHK
        pltpu.sync_copy(v2p_hbm.at[pl.ds(off, n_real)], vb.at[pl.ds(0, n_real)])
        pltpu.sync_copy(bat_hbm.at[pl.ds(off, n_real)], bb.at[pl.ds(0, n_real)])
        if n_real < CHK:
            for u in range((CHK - n_real) // L):
                vb[pl.ds(n_real + u * L, L)] = jnp.zeros((L,), jnp.int32)
                bb[pl.ds(n_real + u * L, L)] = jnp.full((L,), DUMP_B, jnp.int32)

        def fbody(j, carry):
            o = j * L
            v = vb[pl.ds(o, L)]
            b = bb[pl.ds(o, L)]
            fidx[pl.ds(t * CHK + o, L)] = b * N_VOXELS + v
            return carry

        lax.fori_loop(0, CHK // L, fbody, 0)

    for d in zero_descs:
        d.wait()
    plsc.subcore_barrier()

    scat_descs = [
        pltpu.async_copy(ones, cs.at[fidx.at[pl.ds(t * CHK, CHK)]], sem_a, add=True)
        for t in range(NCHK)
    ]
    for d in scat_descs:
        d.wait()

    plsc.subcore_barrier()
    fsems = (sem_f0, sem_f1)
    fly = [None, None]
    for k in range(NFCH):
        n = FCH if k < NFCH - 1 else F_TAIL
        h = k % 2
        if fly[h] is not None:
            fly[h].wait()
        buf = zfb.at[pl.ds(h * FCH, n)]
        pltpu.sync_copy(cs.at[pl.ds(s * F_SLICE + k * FCH, n)], buf)
        fly[h] = pltpu.async_copy(
            buf, out_hbm.at[pl.ds(c * C_BINS + s * F_SLICE + k * FCH, n)],
            fsems[h],
        )
    for d in fly:
        if d is not None:
            d.wait()


def _histogram(v2p_map, batch_idxs):
    mesh = plsc.VectorSubcoreMesh(
        core_axis_name="c", subcore_axis_name="s", num_cores=NC, num_subcores=NS
    )
    return pl.kernel(
        _hist_body,
        out_type=jax.ShapeDtypeStruct((NC * C_BINS,), jnp.float32),
        mesh=mesh,
        scratch_types=[
            pltpu.VMEM((CHK,), jnp.int32),
            pltpu.VMEM((CHK,), jnp.int32),
            pltpu.VMEM((NCHK * CHK,), jnp.int32),
            pltpu.VMEM((CHK,), jnp.float32),
            pltpu.VMEM((ZCH,), jnp.float32),
            pltpu.VMEM_SHARED((C_TOTAL,), jnp.float32),
            pltpu.SemaphoreType.DMA,
            pltpu.SemaphoreType.DMA,
            pltpu.SemaphoreType.DMA,
        ],
    )(v2p_map, batch_idxs)


VSTEP = 2048
NKM = 98304 // VSTEP
V_TAIL = N_VOXELS - NKM * VSTEP
_DN_MK = (((1,), (0,)), ((), ()))
_PREC = lax.Precision.DEFAULT


def _mm_main_body(c_ref, a_ref, acc_ref, cnt_ref):
    i = pl.program_id(0)

    @pl.when(i == 0)
    def _init():
        acc_ref[...] = jnp.zeros_like(acc_ref)
        cnt_ref[...] = jnp.zeros_like(cnt_ref)

    c0 = c_ref[0]
    c1 = c_ref[1]
    ablk = a_ref[...]
    acc_ref[...] += lax.dot_general(
        c0, ablk, _DN_MK, preferred_element_type=jnp.float32, precision=_PREC
    ) + lax.dot_general(
        c1, ablk, _DN_MK, preferred_element_type=jnp.float32, precision=_PREC
    )
    cnt_ref[...] += jnp.sum(c0 + c1, axis=1, keepdims=True)


def _mm_tail_body(ct_ref, at_ref, acc_ref, cnt_ref, cap_ref, ls_ref, out_ref):
    at = at_ref[...]
    sums = acc_ref[...] + lax.dot_general(
        ct_ref[0], at, _DN_MK, preferred_element_type=jnp.float32, precision=_PREC
    ) + lax.dot_general(
        ct_ref[1], at, _DN_MK, preferred_element_type=jnp.float32, precision=_PREC
    )
    cnts = cnt_ref[:, 0:1] + jnp.sum(ct_ref[0] + ct_ref[1], axis=1, keepdims=True)
    pooled = sums / jnp.maximum(cnts, 1.0)
    pn = pooled / jnp.maximum(
        jnp.sqrt(jnp.sum(pooled * pooled, axis=1, keepdims=True)), 1e-12
    )
    cap = cap_ref[...]
    cn = cap / jnp.maximum(
        jnp.sqrt(jnp.sum(cap * cap, axis=1, keepdims=True)), 1e-12
    )
    scale = jnp.exp(ls_ref[0, 0])
    out_ref[...] = (
        lax.dot_general(
            pn, cn, (((1,), (1,)), ((), ())),
            preferred_element_type=jnp.float32, precision=_PREC,
        )
        * scale
    )


def _pool_main(counts_bm, adapter_feats):
    return pl.pallas_call(
        _mm_main_body,
        grid=(NKM,),
        in_specs=[
            pl.BlockSpec((NC, B, VSTEP), lambda i: (0, 0, i)),
            pl.BlockSpec((VSTEP, D), lambda i: (i, 0)),
        ],
        out_specs=[
            pl.BlockSpec((B, D), lambda i: (0, 0)),
            pl.BlockSpec((B, D), lambda i: (0, 0)),
        ],
        out_shape=[
            jax.ShapeDtypeStruct((B, D), jnp.float32),
            jax.ShapeDtypeStruct((B, D), jnp.float32),
        ],
    )(counts_bm, adapter_feats)


def _pool_tail(ct, at, acc, cnt, caption_embed, ls2d):
    return pl.pallas_call(
        _mm_tail_body,
        in_specs=[
            pl.BlockSpec((NC, B, V_TAIL), lambda: (0, 0, 0)),
            pl.BlockSpec((V_TAIL, D), lambda: (0, 0)),
            pl.BlockSpec((B, D), lambda: (0, 0)),
            pl.BlockSpec((B, D), lambda: (0, 0)),
            pl.BlockSpec((B, D), lambda: (0, 0)),
            pl.BlockSpec(memory_space=pltpu.SMEM),
        ],
        out_specs=pl.BlockSpec((B, B), lambda: (0, 0)),
        out_shape=jax.ShapeDtypeStruct((B, B), jnp.float32),
    )(ct, at, acc, cnt, caption_embed, ls2d)


def kernel(adapter_feats, v2p_map, batch_idxs, caption_embed, caption_idx, logit_scale):
    del caption_idx
    counts_bm = _histogram(v2p_map, batch_idxs).reshape(NC, B, N_VOXELS)
    acc, cnt = _pool_main(counts_bm, adapter_feats)
    ct = lax.slice(counts_bm, (0, 0, NKM * VSTEP), (NC, B, N_VOXELS))
    at = lax.slice(adapter_feats, (NKM * VSTEP, 0), (N_VOXELS, D))
    ls2d = jnp.reshape(logit_scale, (1, 1))
    return _pool_tail(ct, at, acc, cnt, caption_embed, ls2d)


# speedup vs baseline: 22.6015x; 1.2234x over previous
import functools

import jax
import jax.numpy as jnp
from jax import lax
from jax.experimental import pallas as pl
from jax.experimental.pallas import tpu as pltpu
from jax.experimental.pallas import tpu_sc as plsc

N_VOXELS = 100000
N_POINTS = 320000
D = 128
B = 16

NC = 2
NS = 16
L = 16
NW = NC * NS

P_TILE = N_POINTS // NW
CH = 128
CHK = 1280
NCHK = -(-P_TILE // CHK)
TAIL = P_TILE - (NCHK - 1) * CHK

C_BINS = B * N_VOXELS
DUMP_B = B
ZCH = 10240
Z_SLICE = 10 * ZCH
C_TOTAL = NS * Z_SLICE
F_SLICE = C_BINS // NS
FCH = 5120
NFCH = -(-F_SLICE // FCH)
F_TAIL = F_SLICE - (NFCH - 1) * FCH


def _hist_body(v2p_hbm, bat_hbm, out_hbm, vb, bb, fidx, ones, zfb, cs,
               sem_a, sem_f0, sem_f1):
    c = lax.axis_index("c")
    s = lax.axis_index("s")
    wid = c * NS + s
    base = wid * P_TILE

    def zbody(i, carry):
        zfb[pl.ds(i * L, L)] = jnp.zeros((L,), jnp.float32)
        return carry

    lax.fori_loop(0, ZCH // L, zbody, 0)
    zero_descs = [
        pltpu.async_copy(zfb, cs.at[pl.ds(s * Z_SLICE + k * ZCH, ZCH)], sem_a)
        for k in range(Z_SLICE // ZCH)
    ]

    def obody(j, carry):
        ones[pl.ds(j * L, L)] = jnp.ones((L,), jnp.float32)
        return carry

    lax.fori_loop(0, CHK // L, obody, 0)

    for t in range(NCHK):
        n_real = CHK if t < NCHK - 1 else TAIL
        off = base + t * CHK
        pltpu.sync_copy(v2p_hbm.at[pl.ds(off, n_real)], vb.at[pl.ds(0, n_real)])
        pltpu.sync_copy(bat_hbm.at[pl.ds(off, n_real)], bb.at[pl.ds(0, n_real)])
        if n_real < CHK:
            for u in range((CHK - n_real) // L):
                vb[pl.ds(n_real + u * L, L)] = jnp.zeros((L,), jnp.int32)
                bb[pl.ds(n_real + u * L, L)] = jnp.full((L,), DUMP_B, jnp.int32)

        def fbody(j, carry):
            o = j * L
            v = vb[pl.ds(o, L)]
            b = bb[pl.ds(o, L)]
            fidx[pl.ds(t * CHK + o, L)] = b * N_VOXELS + v
            return carry

        lax.fori_loop(0, CHK // L, fbody, 0)

    for d in zero_descs:
        d.wait()
    plsc.subcore_barrier()

    scat_descs = [
        pltpu.async_copy(ones, cs.at[fidx.at[pl.ds(t * CHK, CHK)]], sem_a, add=True)
        for t in range(NCHK)
    ]
    for d in scat_descs:
        d.wait()

    plsc.subcore_barrier()
    fsems = (sem_f0, sem_f1)
    fly = [None, None]
    for k in range(NFCH):
        n = FCH if k < NFCH - 1 else F_TAIL
        h = k % 2
        if fly[h] is not None:
            fly[h].wait()
        buf = zfb.at[pl.ds(h * FCH, n)]
        pltpu.sync_copy(cs.at[pl.ds(s * F_SLICE + k * FCH, n)], buf)
        fly[h] = pltpu.async_copy(
            buf, out_hbm.at[pl.ds(c * C_BINS + s * F_SLICE + k * FCH, n)],
            fsems[h],
        )
    for d in fly:
        if d is not None:
            d.wait()


def _histogram(v2p_map, batch_idxs):
    mesh = plsc.VectorSubcoreMesh(
        core_axis_name="c", subcore_axis_name="s", num_cores=NC, num_subcores=NS
    )
    return pl.kernel(
        _hist_body,
        out_type=jax.ShapeDtypeStruct((NC * C_BINS,), jnp.float32),
        mesh=mesh,
        scratch_types=[
            pltpu.VMEM((CHK,), jnp.int32),
            pltpu.VMEM((CHK,), jnp.int32),
            pltpu.VMEM((NCHK * CHK,), jnp.int32),
            pltpu.VMEM((CHK,), jnp.float32),
            pltpu.VMEM((ZCH,), jnp.float32),
            pltpu.VMEM_SHARED((C_TOTAL,), jnp.float32),
            pltpu.SemaphoreType.DMA,
            pltpu.SemaphoreType.DMA,
            pltpu.SemaphoreType.DMA,
        ],
    )(v2p_map, batch_idxs)


VSTEP = 8192
NKM = 98304 // VSTEP
V_TAIL = N_VOXELS - NKM * VSTEP
_DN_MK = (((1,), (0,)), ((), ()))
_PREC = lax.Precision.DEFAULT


def _mm_body(c_ref, a_ref, ct_ref, at_ref, cap_ref, ls_ref, out_ref, acc, cnt):
    i = pl.program_id(0)

    @pl.when(i == 0)
    def _init():
        acc[...] = jnp.zeros_like(acc)
        cnt[...] = jnp.zeros_like(cnt)

    @pl.when(i < NKM)
    def _step():
        c0 = c_ref[0]
        c1 = c_ref[1]
        ablk = a_ref[...]
        acc[...] += lax.dot_general(
            c0, ablk, _DN_MK, preferred_element_type=jnp.float32, precision=_PREC
        ) + lax.dot_general(
            c1, ablk, _DN_MK, preferred_element_type=jnp.float32, precision=_PREC
        )
        cnt[...] += jnp.sum(c0 + c1, axis=1, keepdims=True)

    @pl.when(i == NKM)
    def _tail():
        at = at_ref[...]
        sums = acc[...] + lax.dot_general(
            ct_ref[0], at, _DN_MK, preferred_element_type=jnp.float32,
            precision=_PREC,
        ) + lax.dot_general(
            ct_ref[1], at, _DN_MK, preferred_element_type=jnp.float32,
            precision=_PREC,
        )
        cnts = cnt[:, 0:1] + jnp.sum(
            ct_ref[0] + ct_ref[1], axis=1, keepdims=True
        )
        pooled = sums / jnp.maximum(cnts, 1.0)
        pn = pooled / jnp.maximum(
            jnp.sqrt(jnp.sum(pooled * pooled, axis=1, keepdims=True)), 1e-12
        )
        cap = cap_ref[...]
        cn = cap / jnp.maximum(
            jnp.sqrt(jnp.sum(cap * cap, axis=1, keepdims=True)), 1e-12
        )
        scale = jnp.exp(ls_ref[0, 0])
        out_ref[...] = (
            lax.dot_general(
                pn, cn, (((1,), (1,)), ((), ())),
                preferred_element_type=jnp.float32, precision=_PREC,
            )
            * scale
        )


def _pool_logits(counts_bm, adapter_feats, ct, at, caption_embed, ls2d):
    return pl.pallas_call(
        _mm_body,
        grid=(NKM + 1,),
        in_specs=[
            pl.BlockSpec((NC, B, VSTEP), lambda i: (0, 0, jnp.minimum(i, NKM - 1))),
            pl.BlockSpec((VSTEP, D), lambda i: (jnp.minimum(i, NKM - 1), 0)),
            pl.BlockSpec((NC, B, V_TAIL), lambda i: (0, 0, 0)),
            pl.BlockSpec((V_TAIL, D), lambda i: (0, 0)),
            pl.BlockSpec((B, D), lambda i: (0, 0)),
            pl.BlockSpec(memory_space=pltpu.SMEM),
        ],
        out_specs=pl.BlockSpec((B, B), lambda i: (0, 0)),
        out_shape=jax.ShapeDtypeStruct((B, B), jnp.float32),
        scratch_shapes=[
            pltpu.VMEM((B, D), jnp.float32),
            pltpu.VMEM((B, D), jnp.float32),
        ],
    )(counts_bm, adapter_feats, ct, at, caption_embed, ls2d)


def kernel(adapter_feats, v2p_map, batch_idxs, caption_embed, caption_idx, logit_scale):
    del caption_idx
    counts_bm = _histogram(v2p_map, batch_idxs).reshape(NC, B, N_VOXELS)
    ct = lax.slice(counts_bm, (0, 0, NKM * VSTEP), (NC, B, N_VOXELS))
    at = lax.slice(adapter_feats, (NKM * VSTEP, 0), (N_VOXELS, D))
    ls2d = jnp.reshape(logit_scale, (1, 1))
    return _pool_logits(counts_bm, adapter_feats, ct, at, caption_embed, ls2d)


# speedup vs baseline: 22.7796x; 1.0079x over previous
import functools

import jax
import jax.numpy as jnp
from jax import lax
from jax.experimental import pallas as pl
from jax.experimental.pallas import tpu as pltpu
from jax.experimental.pallas import tpu_sc as plsc

N_VOXELS = 100000
N_POINTS = 320000
D = 128
B = 16

NC = 2
NS = 16
L = 16
NW = NC * NS

P_TILE = N_POINTS // NW
CH = 128
CHK = 1280
NCHK = -(-P_TILE // CHK)
TAIL = P_TILE - (NCHK - 1) * CHK

C_BINS = B * N_VOXELS
DUMP_B = B
ZCH = 10240
Z_SLICE = 10 * ZCH
C_TOTAL = NS * Z_SLICE
F_SLICE = C_BINS // NS
FCH = 5120
NFCH = -(-F_SLICE // FCH)
F_TAIL = F_SLICE - (NFCH - 1) * FCH


def _hist_body(v2p_hbm, bat_hbm, out_hbm, vb, bb, fidx, ones, zfb, cs,
               sem_a, sem_f0, sem_f1):
    c = lax.axis_index("c")
    s = lax.axis_index("s")
    wid = c * NS + s
    base = wid * P_TILE

    def zbody(i, carry):
        zfb[pl.ds(i * L, L)] = jnp.zeros((L,), jnp.float32)
        return carry

    lax.fori_loop(0, ZCH // L, zbody, 0)
    zero_descs = [
        pltpu.async_copy(zfb, cs.at[pl.ds(s * Z_SLICE + k * ZCH, ZCH)], sem_a)
        for k in range(Z_SLICE // ZCH)
    ]

    def obody(j, carry):
        ones[pl.ds(j * L, L)] = jnp.ones((L,), jnp.float32)
        return carry

    lax.fori_loop(0, CHK // L, obody, 0)

    for t in range(NCHK):
        n_real = CHK if t < NCHK - 1 else TAIL
        off = base + t * CHK
        pltpu.sync_copy(v2p_hbm.at[pl.ds(off, n_real)], vb.at[pl.ds(0, n_real)])
        pltpu.sync_copy(bat_hbm.at[pl.ds(off, n_real)], bb.at[pl.ds(0, n_real)])
        if n_real < CHK:
            for u in range((CHK - n_real) // L):
                vb[pl.ds(n_real + u * L, L)] = jnp.zeros((L,), jnp.int32)
                bb[pl.ds(n_real + u * L, L)] = jnp.full((L,), DUMP_B, jnp.int32)

        def fbody(j, carry):
            o = j * L
            v = vb[pl.ds(o, L)]
            b = bb[pl.ds(o, L)]
            fidx[pl.ds(t * CHK + o, L)] = b * N_VOXELS + v
            return carry

        lax.fori_loop(0, CHK // L, fbody, 0)

    for d in zero_descs:
        d.wait()
    plsc.subcore_barrier()

    scat_descs = [
        pltpu.async_copy(ones, cs.at[fidx.at[pl.ds(t * CHK, CHK)]], sem_a, add=True)
        for t in range(NCHK)
    ]
    for d in scat_descs:
        d.wait()

    plsc.subcore_barrier()
    fsems = (sem_f0, sem_f1)
    fly = [None, None]
    for k in range(NFCH):
        n = FCH if k < NFCH - 1 else F_TAIL
        h = k % 2
        if fly[h] is not None:
            fly[h].wait()
        buf = zfb.at[pl.ds(h * FCH, n)]
        pltpu.sync_copy(cs.at[pl.ds(s * F_SLICE + k * FCH, n)], buf)
        fly[h] = pltpu.async_copy(
            buf, out_hbm.at[pl.ds(c * C_BINS + s * F_SLICE + k * FCH, n)],
            fsems[h],
        )
    for d in fly:
        if d is not None:
            d.wait()


def _histogram(v2p_map, batch_idxs):
    mesh = plsc.VectorSubcoreMesh(
        core_axis_name="c", subcore_axis_name="s", num_cores=NC, num_subcores=NS
    )
    return pl.kernel(
        _hist_body,
        out_type=jax.ShapeDtypeStruct((NC * C_BINS,), jnp.float32),
        mesh=mesh,
        scratch_types=[
            pltpu.VMEM((CHK,), jnp.int32),
            pltpu.VMEM((CHK,), jnp.int32),
            pltpu.VMEM((NCHK * CHK,), jnp.int32),
            pltpu.VMEM((CHK,), jnp.float32),
            pltpu.VMEM((ZCH,), jnp.float32),
            pltpu.VMEM_SHARED((C_TOTAL,), jnp.float32),
            pltpu.SemaphoreType.DMA,
            pltpu.SemaphoreType.DMA,
            pltpu.SemaphoreType.DMA,
        ],
    )(v2p_map, batch_idxs)


VSTEP = 16384
NKM = 98304 // VSTEP
V_TAIL = N_VOXELS - NKM * VSTEP
_DN_MK = (((1,), (0,)), ((), ()))
_PREC = lax.Precision.DEFAULT


def _mm_body(c_ref, a_ref, ct_ref, at_ref, cap_ref, ls_ref, out_ref, acc, cnt):
    i = pl.program_id(0)

    @pl.when(i == 0)
    def _init():
        acc[...] = jnp.zeros_like(acc)
        cnt[...] = jnp.zeros_like(cnt)

    @pl.when(i < NKM)
    def _step():
        c0 = c_ref[0]
        c1 = c_ref[1]
        ablk = a_ref[...]
        acc[...] += lax.dot_general(
            c0, ablk, _DN_MK, preferred_element_type=jnp.float32, precision=_PREC
        ) + lax.dot_general(
            c1, ablk, _DN_MK, preferred_element_type=jnp.float32, precision=_PREC
        )
        cnt[...] += jnp.sum(c0 + c1, axis=1, keepdims=True)

    @pl.when(i == NKM)
    def _tail():
        at = at_ref[...]
        sums = acc[...] + lax.dot_general(
            ct_ref[0], at, _DN_MK, preferred_element_type=jnp.float32,
            precision=_PREC,
        ) + lax.dot_general(
            ct_ref[1], at, _DN_MK, preferred_element_type=jnp.float32,
            precision=_PREC,
        )
        cnts = cnt[:, 0:1] + jnp.sum(
            ct_ref[0] + ct_ref[1], axis=1, keepdims=True
        )
        pooled = sums / jnp.maximum(cnts, 1.0)
        pn = pooled / jnp.maximum(
            jnp.sqrt(jnp.sum(pooled * pooled, axis=1, keepdims=True)), 1e-12
        )
        cap = cap_ref[...]
        cn = cap / jnp.maximum(
            jnp.sqrt(jnp.sum(cap * cap, axis=1, keepdims=True)), 1e-12
        )
        scale = jnp.exp(ls_ref[0, 0])
        out_ref[...] = (
            lax.dot_general(
                pn, cn, (((1,), (1,)), ((), ())),
                preferred_element_type=jnp.float32, precision=_PREC,
            )
            * scale
        )


def _pool_logits(counts_bm, adapter_feats, ct, at, caption_embed, ls2d):
    return pl.pallas_call(
        _mm_body,
        grid=(NKM + 1,),
        in_specs=[
            pl.BlockSpec((NC, B, VSTEP), lambda i: (0, 0, jnp.minimum(i, NKM - 1))),
            pl.BlockSpec((VSTEP, D), lambda i: (jnp.minimum(i, NKM - 1), 0)),
            pl.BlockSpec((NC, B, V_TAIL), lambda i: (0, 0, 0)),
            pl.BlockSpec((V_TAIL, D), lambda i: (0, 0)),
            pl.BlockSpec((B, D), lambda i: (0, 0)),
            pl.BlockSpec(memory_space=pltpu.SMEM),
        ],
        out_specs=pl.BlockSpec((B, B), lambda i: (0, 0)),
        out_shape=jax.ShapeDtypeStruct((B, B), jnp.float32),
        scratch_shapes=[
            pltpu.VMEM((B, D), jnp.float32),
            pltpu.VMEM((B, D), jnp.float32),
        ],
    )(counts_bm, adapter_feats, ct, at, caption_embed, ls2d)


def kernel(adapter_feats, v2p_map, batch_idxs, caption_embed, caption_idx, logit_scale):
    del caption_idx
    counts_bm = _histogram(v2p_map, batch_idxs).reshape(NC, B, N_VOXELS)
    ct = lax.slice(counts_bm, (0, 0, NKM * VSTEP), (NC, B, N_VOXELS))
    at = lax.slice(adapter_feats, (NKM * VSTEP, 0), (N_VOXELS, D))
    ls2d = jnp.reshape(logit_scale, (1, 1))
    return _pool_logits(counts_bm, adapter_feats, ct, at, caption_embed, ls2d)


# speedup vs baseline: 25.0714x; 1.1006x over previous
import functools

import jax
import jax.numpy as jnp
from jax import lax
from jax.experimental import pallas as pl
from jax.experimental.pallas import tpu as pltpu
from jax.experimental.pallas import tpu_sc as plsc

N_VOXELS = 100000
N_POINTS = 320000
D = 128
B = 16

NC = 2
NS = 16
L = 16
NW = NC * NS

P_TILE = N_POINTS // NW
CH = 128
CHK = 1280
NCHK = -(-P_TILE // CHK)
TAIL = P_TILE - (NCHK - 1) * CHK

C_BINS = B * N_VOXELS
DUMP_B = B
ZCH = 10240
Z_SLICE = 10 * ZCH
C_TOTAL = NS * Z_SLICE
F_SLICE = C_BINS // NS
FCH = 5120
NFCH = -(-F_SLICE // FCH)
F_TAIL = F_SLICE - (NFCH - 1) * FCH


def _hist_body(v2p_hbm, bat_hbm, out_hbm, vb, bb, fidx, ones, zfb, cs,
               sem_a, sem_f0, sem_f1):
    c = lax.axis_index("c")
    s = lax.axis_index("s")
    wid = c * NS + s
    base = wid * P_TILE
    hsems = (sem_f0, sem_f1)

    def zbody(i, carry):
        zfb[pl.ds(i * L, L)] = jnp.zeros((L,), jnp.float32)
        return carry

    lax.fori_loop(0, ZCH // L, zbody, 0)
    zero_descs = [
        pltpu.async_copy(zfb, cs.at[pl.ds(s * Z_SLICE + k * ZCH, ZCH)], sem_a)
        for k in range(Z_SLICE // ZCH)
    ]

    def obody(j, carry):
        ones[pl.ds(j * L, L)] = jnp.ones((L,), jnp.float32)
        return carry

    lax.fori_loop(0, CHK // L, obody, 0)

    def _fire_loads(t):
        h = t % 2
        n = CHK if t < NCHK - 1 else TAIL
        off = base + t * CHK
        return [
            pltpu.async_copy(
                v2p_hbm.at[pl.ds(off, n)], vb.at[pl.ds(h * CHK, n)], hsems[h]
            ),
            pltpu.async_copy(
                bat_hbm.at[pl.ds(off, n)], bb.at[pl.ds(h * CHK, n)], hsems[h]
            ),
        ]

    pending = _fire_loads(0)
    for t in range(NCHK):
        nxt = _fire_loads(t + 1) if t + 1 < NCHK else None
        for d in pending:
            d.wait()
        h = t % 2
        if t == NCHK - 1:
            for u in range((CHK - TAIL) // L):
                vb[pl.ds(h * CHK + TAIL + u * L, L)] = jnp.zeros((L,), jnp.int32)
                bb[pl.ds(h * CHK + TAIL + u * L, L)] = jnp.full((L,), DUMP_B, jnp.int32)

        def fbody(j, carry):
            o = j * L
            v = vb[pl.ds(h * CHK + o, L)]
            b = bb[pl.ds(h * CHK + o, L)]
            fidx[pl.ds(t * CHK + o, L)] = b * N_VOXELS + v
            return carry

        lax.fori_loop(0, CHK // L, fbody, 0)
        pending = nxt

    for d in zero_descs:
        d.wait()
    plsc.subcore_barrier()

    scat_descs = [
        pltpu.async_copy(ones, cs.at[fidx.at[pl.ds(t * CHK, CHK)]], sem_a, add=True)
        for t in range(NCHK)
    ]
    for d in scat_descs:
        d.wait()

    plsc.subcore_barrier()

    def _n(k):
        return FCH if k < NFCH - 1 else F_TAIL

    hop1 = [None] * NFCH
    hop2 = [None] * NFCH
    for k in range(NFCH + 1):
        if k < NFCH:
            hh = k % 2
            if k >= 2:
                hop2[k - 2].wait()
            hop1[k] = pltpu.async_copy(
                cs.at[pl.ds(s * F_SLICE + k * FCH, _n(k))],
                zfb.at[pl.ds(hh * FCH, _n(k))],
                sem_a,
            )
        if k >= 1:
            hop1[k - 1].wait()
            hh = (k - 1) % 2
            hop2[k - 1] = pltpu.async_copy(
                zfb.at[pl.ds(hh * FCH, _n(k - 1))],
                out_hbm.at[pl.ds(c * C_BINS + s * F_SLICE + (k - 1) * FCH, _n(k - 1))],
                hsems[hh],
            )
    hop2[NFCH - 2].wait()
    hop2[NFCH - 1].wait()


def _histogram(v2p_map, batch_idxs):
    mesh = plsc.VectorSubcoreMesh(
        core_axis_name="c", subcore_axis_name="s", num_cores=NC, num_subcores=NS
    )
    return pl.kernel(
        _hist_body,
        out_type=jax.ShapeDtypeStruct((NC * C_BINS,), jnp.float32),
        mesh=mesh,
        scratch_types=[
            pltpu.VMEM((2 * CHK,), jnp.int32),
            pltpu.VMEM((2 * CHK,), jnp.int32),
            pltpu.VMEM((NCHK * CHK,), jnp.int32),
            pltpu.VMEM((CHK,), jnp.float32),
            pltpu.VMEM((2 * FCH,), jnp.float32),
            pltpu.VMEM_SHARED((C_TOTAL,), jnp.float32),
            pltpu.SemaphoreType.DMA,
            pltpu.SemaphoreType.DMA,
            pltpu.SemaphoreType.DMA,
        ],
    )(v2p_map, batch_idxs)


VSTEP = 16384
NKM = 98304 // VSTEP
V_TAIL = N_VOXELS - NKM * VSTEP
_DN_MK = (((1,), (0,)), ((), ()))
_PREC = lax.Precision.DEFAULT


def _mm_body(c_ref, a_ref, ct_ref, at_ref, cap_ref, ls_ref, out_ref, acc, cnt):
    i = pl.program_id(0)

    @pl.when(i == 0)
    def _init():
        acc[...] = jnp.zeros_like(acc)
        cnt[...] = jnp.zeros_like(cnt)

    @pl.when(i < NKM)
    def _step():
        c0 = c_ref[0]
        c1 = c_ref[1]
        ablk = a_ref[...]
        acc[...] += lax.dot_general(
            c0, ablk, _DN_MK, preferred_element_type=jnp.float32, precision=_PREC
        ) + lax.dot_general(
            c1, ablk, _DN_MK, preferred_element_type=jnp.float32, precision=_PREC
        )
        cnt[...] += jnp.sum(c0 + c1, axis=1, keepdims=True)

    @pl.when(i == NKM)
    def _tail():
        at = at_ref[...]
        sums = acc[...] + lax.dot_general(
            ct_ref[0], at, _DN_MK, preferred_element_type=jnp.float32,
            precision=_PREC,
        ) + lax.dot_general(
            ct_ref[1], at, _DN_MK, preferred_element_type=jnp.float32,
            precision=_PREC,
        )
        cnts = cnt[:, 0:1] + jnp.sum(
            ct_ref[0] + ct_ref[1], axis=1, keepdims=True
        )
        pooled = sums / jnp.maximum(cnts, 1.0)
        pn = pooled / jnp.maximum(
            jnp.sqrt(jnp.sum(pooled * pooled, axis=1, keepdims=True)), 1e-12
        )
        cap = cap_ref[...]
        cn = cap / jnp.maximum(
            jnp.sqrt(jnp.sum(cap * cap, axis=1, keepdims=True)), 1e-12
        )
        scale = jnp.exp(ls_ref[0, 0])
        out_ref[...] = (
            lax.dot_general(
                pn, cn, (((1,), (1,)), ((), ())),
                preferred_element_type=jnp.float32, precision=_PREC,
            )
            * scale
        )


def _pool_logits(counts_bm, adapter_feats, ct, at, caption_embed, ls2d):
    return pl.pallas_call(
        _mm_body,
        grid=(NKM + 1,),
        in_specs=[
            pl.BlockSpec((NC, B, VSTEP), lambda i: (0, 0, jnp.minimum(i, NKM - 1))),
            pl.BlockSpec((VSTEP, D), lambda i: (jnp.minimum(i, NKM - 1), 0)),
            pl.BlockSpec((NC, B, V_TAIL), lambda i: (0, 0, 0)),
            pl.BlockSpec((V_TAIL, D), lambda i: (0, 0)),
            pl.BlockSpec((B, D), lambda i: (0, 0)),
            pl.BlockSpec(memory_space=pltpu.SMEM),
        ],
        out_specs=pl.BlockSpec((B, B), lambda i: (0, 0)),
        out_shape=jax.ShapeDtypeStruct((B, B), jnp.float32),
        scratch_shapes=[
            pltpu.VMEM((B, D), jnp.float32),
            pltpu.VMEM((B, D), jnp.float32),
        ],
    )(counts_bm, adapter_feats, ct, at, caption_embed, ls2d)


def kernel(adapter_feats, v2p_map, batch_idxs, caption_embed, caption_idx, logit_scale):
    del caption_idx
    counts_bm = _histogram(v2p_map, batch_idxs).reshape(NC, B, N_VOXELS)
    ct = lax.slice(counts_bm, (0, 0, NKM * VSTEP), (NC, B, N_VOXELS))
    at = lax.slice(adapter_feats, (NKM * VSTEP, 0), (N_VOXELS, D))
    ls2d = jnp.reshape(logit_scale, (1, 1))
    return _pool_logits(counts_bm, adapter_feats, ct, at, caption_embed, ls2d)
